# baseline (speedup 1.0000x reference)
import functools

import jax
import jax.numpy as jnp
from jax import lax
from jax.experimental import pallas as pl
from jax.experimental.pallas import tpu as pltpu
from jax.experimental.pallas import tpu_sc as plsc

NUM_VARS = 26
VOCAB = 100002
DIM = 32
B = 16384
L = 20
N = B * L
NW = 32
ROWS_W = N // NW
C = 1024
NCHUNK = ROWS_W // C
LANES = 16
R = 2

_mesh = plsc.VectorSubcoreMesh(core_axis_name="c", subcore_axis_name="s")


@functools.partial(
    pl.kernel,
    mesh=_mesh,
    out_type=jax.ShapeDtypeStruct((N, NUM_VARS * DIM), jnp.float32),
    scratch_types=[
        pltpu.VMEM((R, C), jnp.int32),
        pltpu.VMEM((R, C, DIM), jnp.float32),
        pltpu.SemaphoreType.DMA((R,)),
        pltpu.SemaphoreType.DMA((R,)),
    ],
    compiler_params=pltpu.CompilerParams(use_tc_tiling_on_sc=False,
                                         needs_layout_passes=False),
)
def _bank(*refs):
    xidx = refs[0]
    tables = refs[1:1 + NUM_VARS]
    out = refs[1 + NUM_VARS]
    idx_v, rows_v, gsem, ssem = refs[2 + NUM_VARS:]

    wid = lax.axis_index("s") * 2 + lax.axis_index("c")
    wbase = wid * ROWS_W

    def chunk_body(ci, _):
        base = wbase + ci * C

        def load_clamp_gather(s):
            r = s % R
            pltpu.sync_copy(xidx.at[s, pl.ds(base, C)], idx_v.at[r])

            def clamp_body(j, _):
                v = idx_v[r, pl.ds(j * LANES, LANES)]
                v = jnp.where(v == -1, VOCAB - 1, v)
                v = jnp.minimum(jnp.maximum(v, 0), VOCAB - 1)
                idx_v[r, pl.ds(j * LANES, LANES)] = v
                return _

            lax.fori_loop(0, C // LANES, clamp_body, None)
            pltpu.async_copy(tables[s].at[idx_v.at[r]], rows_v.at[r],
                             gsem.at[r])

        def scatter(s):
            r = s % R
            pltpu.make_async_copy(tables[s].at[idx_v.at[r]], rows_v.at[r],
                                  gsem.at[r]).wait()
            pltpu.async_copy(rows_v.at[r],
                             out.at[pl.ds(base, C), pl.ds(s * DIM, DIM)],
                             ssem.at[r])

        def drain_scatter(s):
            r = s % R
            pltpu.make_async_copy(rows_v.at[r],
                                  out.at[pl.ds(base, C), pl.ds(s * DIM, DIM)],
                                  ssem.at[r]).wait()

        for s in range(NUM_VARS):
            if s >= R:
                drain_scatter(s - R)
            load_clamp_gather(s)
            if s >= 1:
                scatter(s - 1)
        scatter(NUM_VARS - 1)
        drain_scatter(NUM_VARS - 2)
        drain_scatter(NUM_VARS - 1)
        return _

    lax.fori_loop(0, NCHUNK, chunk_body, None)


def kernel(inputs_0, inputs_1, inputs_2, inputs_3, inputs_4, inputs_5, inputs_6, inputs_7, inputs_8, inputs_9, inputs_10, inputs_11, inputs_12, inputs_13, inputs_14, inputs_15, inputs_16, inputs_17, inputs_18, inputs_19, inputs_20, inputs_21, inputs_22, inputs_23, inputs_24, inputs_25, table_0, table_1, table_2, table_3, table_4, table_5, table_6, table_7, table_8, table_9, table_10, table_11, table_12, table_13, table_14, table_15, table_16, table_17, table_18, table_19, table_20, table_21, table_22, table_23, table_24, table_25):
    args = locals()
    ins = [args[f"inputs_{i}"] for i in range(NUM_VARS)]
    tabs = [args[f"table_{i}"] for i in range(NUM_VARS)]
    xidx = jnp.stack(ins).reshape(NUM_VARS, N)
    out = _bank(xidx, *tabs)
    return out.reshape(B, L, NUM_VARS * DIM)


# speedup vs baseline: 1.0670x; 1.0670x over previous
import functools

import jax
import jax.numpy as jnp
from jax import lax
from jax.experimental import pallas as pl
from jax.experimental.pallas import tpu as pltpu
from jax.experimental.pallas import tpu_sc as plsc

NUM_VARS = 26
VOCAB = 100002
DIM = 32
B = 16384
L = 20
N = B * L
NW = 32
ROWS_W = N // NW
C = 1024
NCHUNK = ROWS_W // C
LANES = 16
R = 2

_mesh = plsc.VectorSubcoreMesh(core_axis_name="c", subcore_axis_name="s")


@functools.partial(
    pl.kernel,
    mesh=_mesh,
    out_type=jax.ShapeDtypeStruct((N, NUM_VARS * DIM), jnp.float32),
    scratch_types=[
        pltpu.VMEM((R, C), jnp.int32),
        pltpu.VMEM((R, C, DIM), jnp.float32),
        pltpu.SemaphoreType.DMA((R,)),
        pltpu.SemaphoreType.DMA((R,)),
    ],
    compiler_params=pltpu.CompilerParams(use_tc_tiling_on_sc=False,
                                         needs_layout_passes=False),
)
def _bank(*refs):
    inputs = refs[:NUM_VARS]
    tables = refs[NUM_VARS:2 * NUM_VARS]
    out = refs[2 * NUM_VARS]
    idx_v, rows_v, gsem, ssem = refs[2 * NUM_VARS + 1:]

    wid = lax.axis_index("s") * 2 + lax.axis_index("c")
    wbase = wid * ROWS_W

    def chunk_body(ci, _):
        base = wbase + ci * C

        def load_clamp_gather(s):
            r = s % R
            pltpu.sync_copy(inputs[s].at[pl.ds(base, C)], idx_v.at[r])

            def clamp_body(j, _):
                v = idx_v[r, pl.ds(j * LANES, LANES)]
                v = jnp.where(v == -1, VOCAB - 1, v)
                v = jnp.minimum(jnp.maximum(v, 0), VOCAB - 1)
                idx_v[r, pl.ds(j * LANES, LANES)] = v
                return _

            lax.fori_loop(0, C // LANES, clamp_body, None)
            pltpu.async_copy(tables[s].at[idx_v.at[r]], rows_v.at[r],
                             gsem.at[r])

        def scatter(s):
            r = s % R
            pltpu.make_async_copy(tables[s].at[idx_v.at[r]], rows_v.at[r],
                                  gsem.at[r]).wait()
            pltpu.async_copy(rows_v.at[r],
                             out.at[pl.ds(base, C), pl.ds(s * DIM, DIM)],
                             ssem.at[r])

        def drain_scatter(s):
            r = s % R
            pltpu.make_async_copy(rows_v.at[r],
                                  out.at[pl.ds(base, C), pl.ds(s * DIM, DIM)],
                                  ssem.at[r]).wait()

        for s in range(NUM_VARS):
            if s >= R:
                drain_scatter(s - R)
            load_clamp_gather(s)
            if s >= 1:
                scatter(s - 1)
        scatter(NUM_VARS - 1)
        drain_scatter(NUM_VARS - 2)
        drain_scatter(NUM_VARS - 1)
        return _

    lax.fori_loop(0, NCHUNK, chunk_body, None)


def kernel(inputs_0, inputs_1, inputs_2, inputs_3, inputs_4, inputs_5, inputs_6, inputs_7, inputs_8, inputs_9, inputs_10, inputs_11, inputs_12, inputs_13, inputs_14, inputs_15, inputs_16, inputs_17, inputs_18, inputs_19, inputs_20, inputs_21, inputs_22, inputs_23, inputs_24, inputs_25, table_0, table_1, table_2, table_3, table_4, table_5, table_6, table_7, table_8, table_9, table_10, table_11, table_12, table_13, table_14, table_15, table_16, table_17, table_18, table_19, table_20, table_21, table_22, table_23, table_24, table_25):
    args = locals()
    ins = [args[f"inputs_{i}"] for i in range(NUM_VARS)]
    tabs = [args[f"table_{i}"] for i in range(NUM_VARS)]
    flats = [x.reshape(N) for x in ins]
    out = _bank(*flats, *tabs)
    return out.reshape(B, L, NUM_VARS * DIM)


# speedup vs baseline: 1.1116x; 1.0418x over previous
import functools

import jax
import jax.numpy as jnp
from jax import lax
from jax.experimental import pallas as pl
from jax.experimental.pallas import tpu as pltpu
from jax.experimental.pallas import tpu_sc as plsc

NUM_VARS = 26
VOCAB = 100002
DIM = 32
B = 16384
L = 20
N = B * L
NW = 32
ROWS_W = N // NW
C = 1024
NCHUNK = ROWS_W // C
LANES = 16
R = 3

_mesh = plsc.VectorSubcoreMesh(core_axis_name="c", subcore_axis_name="s")


@functools.partial(
    pl.kernel,
    mesh=_mesh,
    out_type=jax.ShapeDtypeStruct((N, NUM_VARS * DIM), jnp.float32),
    scratch_types=[
        pltpu.VMEM((R, C), jnp.int32),
        pltpu.VMEM((R, C, DIM), jnp.float32),
        pltpu.SemaphoreType.DMA((R,)),
        pltpu.SemaphoreType.DMA((R,)),
    ],
    compiler_params=pltpu.CompilerParams(use_tc_tiling_on_sc=False,
                                         needs_layout_passes=False),
)
def _bank(*refs):
    inputs = refs[:NUM_VARS]
    tables = refs[NUM_VARS:2 * NUM_VARS]
    out = refs[2 * NUM_VARS]
    idx_v, rows_v, gsem, ssem = refs[2 * NUM_VARS + 1:]

    wid = lax.axis_index("s") * 2 + lax.axis_index("c")
    wbase = wid * ROWS_W

    def chunk_body(ci, _):
        base = wbase + ci * C

        def load_clamp_gather(s):
            r = s % R
            pltpu.sync_copy(inputs[s].at[pl.ds(base, C)], idx_v.at[r])

            def clamp_body(j, _):
                v = idx_v[r, pl.ds(j * LANES, LANES)]
                v = jnp.where(v == -1, VOCAB - 1, v)
                v = jnp.minimum(jnp.maximum(v, 0), VOCAB - 1)
                idx_v[r, pl.ds(j * LANES, LANES)] = v
                return _

            lax.fori_loop(0, C // LANES, clamp_body, None)
            pltpu.async_copy(tables[s].at[idx_v.at[r]], rows_v.at[r],
                             gsem.at[r])

        def scatter(s):
            r = s % R
            pltpu.make_async_copy(tables[s].at[idx_v.at[r]], rows_v.at[r],
                                  gsem.at[r]).wait()
            pltpu.async_copy(rows_v.at[r],
                             out.at[pl.ds(base, C), pl.ds(s * DIM, DIM)],
                             ssem.at[r])

        def drain_scatter(s):
            r = s % R
            pltpu.make_async_copy(rows_v.at[r],
                                  out.at[pl.ds(base, C), pl.ds(s * DIM, DIM)],
                                  ssem.at[r]).wait()

        for s in range(NUM_VARS):
            if s >= R:
                drain_scatter(s - R)
            load_clamp_gather(s)
            if s >= 1:
                scatter(s - 1)
        scatter(NUM_VARS - 1)
        for s in range(NUM_VARS - R, NUM_VARS):
            drain_scatter(s)
        return _

    lax.fori_loop(0, NCHUNK, chunk_body, None)


def kernel(inputs_0, inputs_1, inputs_2, inputs_3, inputs_4, inputs_5, inputs_6, inputs_7, inputs_8, inputs_9, inputs_10, inputs_11, inputs_12, inputs_13, inputs_14, inputs_15, inputs_16, inputs_17, inputs_18, inputs_19, inputs_20, inputs_21, inputs_22, inputs_23, inputs_24, inputs_25, table_0, table_1, table_2, table_3, table_4, table_5, table_6, table_7, table_8, table_9, table_10, table_11, table_12, table_13, table_14, table_15, table_16, table_17, table_18, table_19, table_20, table_21, table_22, table_23, table_24, table_25):
    args = locals()
    ins = [args[f"inputs_{i}"] for i in range(NUM_VARS)]
    tabs = [args[f"table_{i}"] for i in range(NUM_VARS)]
    flats = [x.reshape(N) for x in ins]
    out = _bank(*flats, *tabs)
    return out.reshape(B, L, NUM_VARS * DIM)
